# baseline (speedup 1.0000x reference)
import jax
import jax.numpy as jnp
import numpy as np
from jax import lax
from jax.experimental import pallas as pl
from jax.experimental.pallas import tpu as pltpu
from jax.experimental.pallas import tpu_sc as plsc

_VOCAB = 1000000
_D = 64
_BATCH = 4096
_SEQ = 200
_POS_LEN = 2048

_NC = 2
_NS = 16
_NW = _NC * _NS
_LANES = 16

_N_ROWS = _BATCH * _SEQ
_ROWS_PER_W = _N_ROWS // _NW
_CH = 800
_G = 100
_N_GATHERS = _CH // _G
_N_CHUNKS = _ROWS_PER_W // _CH


def _pos_encoding_np(length, depth):
    half = depth / 2
    positions = np.arange(length)[:, np.newaxis]
    depths = np.arange(half)[np.newaxis, :] / half
    angle_rates = 1 / 10000 ** depths
    angle_rads = positions * angle_rates
    return np.concatenate(
        [np.sin(angle_rads), np.cos(angle_rads)], axis=-1
    ).astype(np.float32)


_POS = _pos_encoding_np(_POS_LEN, _D)[:_SEQ]


def _sc_body(table_hbm, idx_hbm, pos_hbm, out_hbm,
             idx_v, gbuf, posbuf, gsem):
    wid = lax.axis_index("s") * _NC + lax.axis_index("c")
    base = wid * _ROWS_PER_W

    for t in range(_CH // _SEQ):
        pltpu.sync_copy(pos_hbm, posbuf.at[pl.ds(t * _SEQ, _SEQ)])

    def chunk_body(c, carry):
        row0 = pl.multiple_of(base + c * _CH, _CH)
        irow0 = pl.multiple_of(row0 // _G, _N_GATHERS)
        pltpu.sync_copy(idx_hbm.at[pl.ds(irow0, _N_GATHERS)], idx_v)
        for j in range(_N_GATHERS):
            pltpu.async_copy(
                table_hbm.at[idx_v.at[j]],
                gbuf.at[pl.ds(j * _G, _G)],
                gsem,
            )
        for j in range(_N_GATHERS):
            pltpu.make_async_copy(
                table_hbm.at[idx_v.at[j]],
                gbuf.at[pl.ds(j * _G, _G)],
                gsem,
            ).wait()

        def vec_body(r, carry2):
            for d in range(_D // _LANES):
                col = d * _LANES
                g = gbuf[r, pl.ds(col, _LANES)]
                p = posbuf[r, pl.ds(col, _LANES)]
                gbuf[r, pl.ds(col, _LANES)] = g * 8.0 + p
            return carry2

        lax.fori_loop(0, _CH, vec_body, 0, unroll=2)

        pltpu.sync_copy(gbuf, out_hbm.at[pl.ds(row0, _CH)])
        return carry

    lax.fori_loop(0, _N_CHUNKS, chunk_body, 0)


@jax.jit
def _sc_call(table, xidx, pos):
    mesh = plsc.VectorSubcoreMesh(
        core_axis_name="c", subcore_axis_name="s"
    )
    kfn = pl.kernel(
        _sc_body,
        out_type=jax.ShapeDtypeStruct((_N_ROWS, _D), jnp.float32),
        mesh=mesh,
        scratch_types=[
            pltpu.VMEM((_N_GATHERS, _G), jnp.int32),
            pltpu.VMEM((_CH, _D), jnp.float32),
            pltpu.VMEM((_CH, _D), jnp.float32),
            pltpu.SemaphoreType.DMA,
        ],
        compiler_params=pltpu.CompilerParams(use_tc_tiling_on_sc=False),
    )
    return kfn(table, xidx, pos)


def kernel(x, table):
    xidx = x.reshape((_N_ROWS // _G, _G))
    pos = jnp.asarray(_POS)
    out = _sc_call(table, xidx, pos)
    return out.reshape((_BATCH, _SEQ, _D))


# speedup vs baseline: 1.0928x; 1.0928x over previous
import jax
import jax.numpy as jnp
import numpy as np
from jax import lax
from jax.experimental import pallas as pl
from jax.experimental.pallas import tpu as pltpu
from jax.experimental.pallas import tpu_sc as plsc

_VOCAB = 1000000
_D = 64
_BATCH = 4096
_SEQ = 200
_POS_LEN = 2048

_NC = 2
_NS = 16
_NW = _NC * _NS
_LANES = 16

_N_ROWS = _BATCH * _SEQ
_ROWS_PER_W = _N_ROWS // _NW
_CH = 400
_G = 100
_NG = _CH // _G
_N_CHUNKS = _ROWS_PER_W // _CH
_NBUF = 4
_IROWS_PER_W = _ROWS_PER_W // _G


def _pos_encoding_np(length, depth):
    half = depth / 2
    positions = np.arange(length)[:, np.newaxis]
    depths = np.arange(half)[np.newaxis, :] / half
    angle_rates = 1 / 10000 ** depths
    angle_rads = positions * angle_rates
    return np.concatenate(
        [np.sin(angle_rads), np.cos(angle_rads)], axis=-1
    ).astype(np.float32)


_POS = _pos_encoding_np(_POS_LEN, _D)[:_SEQ]


def _sc_body(table_hbm, idx_hbm, pos_hbm, out_hbm,
             idx0, idx1, g0, g1, g2, g3, posbuf,
             gs0, gs1, gs2, gs3, ss0, ss1, ss2, ss3):
    gbufs = (g0, g1, g2, g3)
    gsems = (gs0, gs1, gs2, gs3)
    ssems = (ss0, ss1, ss2, ss3)
    idxbufs = (idx0, idx1)

    wid = lax.axis_index("s") * _NC + lax.axis_index("c")
    base = wid * _ROWS_PER_W
    ibase = wid * _IROWS_PER_W

    pltpu.sync_copy(pos_hbm, posbuf)


    def do_idx_copy(c_val, kpar):
        pltpu.sync_copy(
            idx_hbm.at[pl.ds(ibase + c_val * _NG, 2 * _NG)],
            idxbufs[kpar],
        )

    def do_fire(c_val, p, kpar, half):
        ib = idxbufs[kpar]
        gb = gbufs[p]
        for j in range(_NG):
            pltpu.async_copy(
                table_hbm.at[ib.at[half * _NG + j]],
                gb.at[pl.ds(j * _G, _G)],
                gsems[p],
            )

    def wait_gathers(p):
        gb = gbufs[p]
        for j in range(_NG):
            pltpu.make_async_copy(
                table_hbm.at[idxbufs[0].at[j]],
                gb.at[pl.ds(j * _G, _G)],
                gsems[p],
            ).wait()

    def compute(p):
        gb = gbufs[p]

        def vec_body(r, carry):
            for q in range(_CH // _SEQ):
                row = q * _SEQ + r
                for d in range(_D // _LANES):
                    col = d * _LANES
                    g = gb[row, pl.ds(col, _LANES)]
                    pv = posbuf[r, pl.ds(col, _LANES)]
                    gb[row, pl.ds(col, _LANES)] = g * 8.0 + pv
            return carry

        lax.fori_loop(0, _SEQ, vec_body, 0, unroll=2)

    def fire_scatter(c_val, p):
        row0 = base + c_val * _CH
        pltpu.async_copy(gbufs[p], out_hbm.at[pl.ds(row0, _CH)], ssems[p])

    def wait_scatter(p):
        pltpu.make_async_copy(
            gbufs[p], out_hbm.at[pl.ds(base, _CH)], ssems[p]
        ).wait()

    def chunk_step(c_val, p, kpar, half, *, fire_next, wait_ssem):
        wait_gathers(p)
        compute(p)
        fire_scatter(c_val, p)
        if fire_next:
            cn = c_val + 2
            pn = (p + 2) % _NBUF
            kn = (kpar + 1) % 2
            if wait_ssem:
                wait_scatter(pn)
            if half == 0:
                do_idx_copy(cn, kn)
            do_fire(cn, pn, kn, half)

    do_idx_copy(0, 0)
    do_fire(0, 0, 0, 0)
    do_fire(1, 1, 0, 1)

    chunk_step(0, 0, 0, 0, fire_next=True, wait_ssem=False)
    chunk_step(1, 1, 0, 1, fire_next=True, wait_ssem=False)
    chunk_step(2, 2, 1, 0, fire_next=True, wait_ssem=True)
    chunk_step(3, 3, 1, 1, fire_next=True, wait_ssem=True)

    def super_body(i, carry):
        c0 = i * _NBUF
        for p in range(_NBUF):
            kpar = (p // 2) % 2
            chunk_step(c0 + p, p, kpar, p % 2,
                       fire_next=True, wait_ssem=True)
        return carry

    lax.fori_loop(1, _N_CHUNKS // _NBUF - 1, super_body, 0)

    c0 = _N_CHUNKS - _NBUF
    chunk_step(c0 + 0, 0, 0, 0, fire_next=True, wait_ssem=True)
    chunk_step(c0 + 1, 1, 0, 1, fire_next=True, wait_ssem=True)
    chunk_step(c0 + 2, 2, 1, 0, fire_next=False, wait_ssem=False)
    chunk_step(c0 + 3, 3, 1, 1, fire_next=False, wait_ssem=False)

    for p in range(_NBUF):
        wait_scatter(p)


@jax.jit
def _sc_call(table, xidx, pos):
    mesh = plsc.VectorSubcoreMesh(
        core_axis_name="c", subcore_axis_name="s"
    )
    kfn = pl.kernel(
        _sc_body,
        out_type=jax.ShapeDtypeStruct((_N_ROWS, _D), jnp.float32),
        mesh=mesh,
        scratch_types=[
            pltpu.VMEM((2 * _NG, _G), jnp.int32),
            pltpu.VMEM((2 * _NG, _G), jnp.int32),
            pltpu.VMEM((_CH, _D), jnp.float32),
            pltpu.VMEM((_CH, _D), jnp.float32),
            pltpu.VMEM((_CH, _D), jnp.float32),
            pltpu.VMEM((_CH, _D), jnp.float32),
            pltpu.VMEM((_SEQ, _D), jnp.float32),
            pltpu.SemaphoreType.DMA,
            pltpu.SemaphoreType.DMA,
            pltpu.SemaphoreType.DMA,
            pltpu.SemaphoreType.DMA,
            pltpu.SemaphoreType.DMA,
            pltpu.SemaphoreType.DMA,
            pltpu.SemaphoreType.DMA,
            pltpu.SemaphoreType.DMA,
        ],
        compiler_params=pltpu.CompilerParams(use_tc_tiling_on_sc=False),
    )
    return kfn(table, xidx, pos)


def kernel(x, table):
    xidx = x.reshape((_N_ROWS // _G, _G))
    pos = jnp.asarray(_POS)
    out = _sc_call(table, xidx, pos)
    return out.reshape((_BATCH, _SEQ, _D))


# speedup vs baseline: 1.3486x; 1.2341x over previous
import jax
import jax.numpy as jnp
import numpy as np
from jax import lax
from jax.experimental import pallas as pl
from jax.experimental.pallas import tpu as pltpu
from jax.experimental.pallas import tpu_sc as plsc

_VOCAB = 1000000
_D = 64
_BATCH = 4096
_SEQ = 200
_POS_LEN = 2048

_NC = 2
_NS = 16
_NW = _NC * _NS
_LANES = 16

_N_ROWS = _BATCH * _SEQ
_ROWS_PER_W = _N_ROWS // _NW
_CH = 400
_G = 100
_NG = _CH // _G
_N_CHUNKS = _ROWS_PER_W // _CH
_NBUF = 4
_IROWS_PER_W = _ROWS_PER_W // _G


def _pos_encoding_np(length, depth):
    half = depth / 2
    positions = np.arange(length)[:, np.newaxis]
    depths = np.arange(half)[np.newaxis, :] / half
    angle_rates = 1 / 10000 ** depths
    angle_rads = positions * angle_rates
    return np.concatenate(
        [np.sin(angle_rads), np.cos(angle_rads)], axis=-1
    ).astype(np.float32)


_POS = _pos_encoding_np(_POS_LEN, _D)[:_SEQ]


def _sc_body(table_hbm, idx_hbm, pos_hbm, out_hbm,
             idx0, idx1, g0, g1, g2, g3, posbuf,
             gs0, gs1, gs2, gs3, ss0, ss1, ss2, ss3):
    gbufs = (g0, g1, g2, g3)
    gsems = (gs0, gs1, gs2, gs3)
    ssems = (ss0, ss1, ss2, ss3)
    idxbufs = (idx0, idx1)

    wid = lax.axis_index("s") * _NC + lax.axis_index("c")
    bbase = wid * (_BATCH // _NW)
    ibase = wid * _IROWS_PER_W

    pltpu.sync_copy(pos_hbm, posbuf)


    def do_idx_copy(c_val, kpar):
        pltpu.sync_copy(
            idx_hbm.at[pl.ds(ibase + c_val * _NG, 2 * _NG)],
            idxbufs[kpar],
        )

    def do_fire(c_val, p, kpar, half):
        ib = idxbufs[kpar]
        gb = gbufs[p]
        for j in range(_NG):
            pltpu.async_copy(
                table_hbm.at[ib.at[half * _NG + j]],
                gb.at[j // 2, pl.ds((j % 2) * _G, _G)],
                gsems[p],
            )

    def wait_gathers(p):
        gb = gbufs[p]
        for j in range(_NG):
            pltpu.make_async_copy(
                table_hbm.at[idxbufs[0].at[j]],
                gb.at[j // 2, pl.ds((j % 2) * _G, _G)],
                gsems[p],
            ).wait()

    def compute(p):
        gb = gbufs[p]

        def vec_body(r, carry):
            for q in range(_CH // _SEQ):
                for d in range(_D // _LANES):
                    col = d * _LANES
                    g = gb[q, r, pl.ds(col, _LANES)]
                    pv = posbuf[r, pl.ds(col, _LANES)]
                    gb[q, r, pl.ds(col, _LANES)] = g * 8.0 + pv
            return carry

        lax.fori_loop(0, _SEQ, vec_body, 0, unroll=2)

    def fire_scatter(c_val, p):
        b0 = bbase + c_val * (_CH // _SEQ)
        pltpu.async_copy(
            gbufs[p],
            out_hbm.at[pl.ds(b0, _CH // _SEQ), :, pl.ds(0, _D)],
            ssems[p])

    def wait_scatter(p):
        pltpu.make_async_copy(
            gbufs[p],
            out_hbm.at[pl.ds(bbase, _CH // _SEQ), :, pl.ds(0, _D)],
            ssems[p]
        ).wait()

    def chunk_step(c_val, p, kpar, half, *, fire_next, wait_ssem):
        wait_gathers(p)
        compute(p)
        fire_scatter(c_val, p)
        if fire_next:
            cn = c_val + 2
            pn = (p + 2) % _NBUF
            kn = (kpar + 1) % 2
            if wait_ssem:
                wait_scatter(pn)
            if half == 0:
                do_idx_copy(cn, kn)
            do_fire(cn, pn, kn, half)

    do_idx_copy(0, 0)
    do_fire(0, 0, 0, 0)
    do_fire(1, 1, 0, 1)

    chunk_step(0, 0, 0, 0, fire_next=True, wait_ssem=False)
    chunk_step(1, 1, 0, 1, fire_next=True, wait_ssem=False)
    chunk_step(2, 2, 1, 0, fire_next=True, wait_ssem=True)
    chunk_step(3, 3, 1, 1, fire_next=True, wait_ssem=True)

    def super_body(i, carry):
        c0 = i * _NBUF
        for p in range(_NBUF):
            kpar = (p // 2) % 2
            chunk_step(c0 + p, p, kpar, p % 2,
                       fire_next=True, wait_ssem=True)
        return carry

    lax.fori_loop(1, _N_CHUNKS // _NBUF - 1, super_body, 0)

    c0 = _N_CHUNKS - _NBUF
    chunk_step(c0 + 0, 0, 0, 0, fire_next=True, wait_ssem=True)
    chunk_step(c0 + 1, 1, 0, 1, fire_next=True, wait_ssem=True)
    chunk_step(c0 + 2, 2, 1, 0, fire_next=False, wait_ssem=False)
    chunk_step(c0 + 3, 3, 1, 1, fire_next=False, wait_ssem=False)

    for p in range(_NBUF):
        wait_scatter(p)


@jax.jit
def _sc_call(table, xidx, pos):
    mesh = plsc.VectorSubcoreMesh(
        core_axis_name="c", subcore_axis_name="s"
    )
    kfn = pl.kernel(
        _sc_body,
        out_type=jax.ShapeDtypeStruct((_BATCH, _SEQ, 2 * _D), jnp.float32),
        mesh=mesh,
        scratch_types=[
            pltpu.VMEM((2 * _NG, _G), jnp.int32),
            pltpu.VMEM((2 * _NG, _G), jnp.int32),
            pltpu.VMEM((2, _SEQ, _D), jnp.float32),
            pltpu.VMEM((2, _SEQ, _D), jnp.float32),
            pltpu.VMEM((2, _SEQ, _D), jnp.float32),
            pltpu.VMEM((2, _SEQ, _D), jnp.float32),
            pltpu.VMEM((_SEQ, _D), jnp.float32),
            pltpu.SemaphoreType.DMA,
            pltpu.SemaphoreType.DMA,
            pltpu.SemaphoreType.DMA,
            pltpu.SemaphoreType.DMA,
            pltpu.SemaphoreType.DMA,
            pltpu.SemaphoreType.DMA,
            pltpu.SemaphoreType.DMA,
            pltpu.SemaphoreType.DMA,
        ],
        compiler_params=pltpu.CompilerParams(use_tc_tiling_on_sc=False),
    )
    return kfn(table, xidx, pos)


def kernel(x, table):
    xidx = x.reshape((_N_ROWS // _G, _G))
    pos = jnp.asarray(_POS)
    out = _sc_call(table, xidx, pos)
    return out[:, :, :_D]


# speedup vs baseline: 1.8601x; 1.3793x over previous
import jax
import jax.numpy as jnp
import numpy as np
from jax import lax
from jax.experimental import pallas as pl
from jax.experimental.pallas import tpu as pltpu
from jax.experimental.pallas import tpu_sc as plsc

_VOCAB = 1000000
_D = 64
_BATCH = 4096
_SEQ = 200
_POS_LEN = 2048

_NC = 2
_NS = 16
_NW = _NC * _NS
_LANES = 16

_N_ROWS = _BATCH * _SEQ
_ROWS_PER_W = _N_ROWS // _NW
_CH = 400
_G = 100
_NG = _CH // _G
_N_CHUNKS = _ROWS_PER_W // _CH
_NBUF = 4
_IROWS_PER_W = _ROWS_PER_W // _G


def _pos_encoding_np(length, depth):
    half = depth / 2
    positions = np.arange(length)[:, np.newaxis]
    depths = np.arange(half)[np.newaxis, :] / half
    angle_rates = 1 / 10000 ** depths
    angle_rads = positions * angle_rates
    return np.concatenate(
        [np.sin(angle_rads), np.cos(angle_rads)], axis=-1
    ).astype(np.float32)


_POS = _pos_encoding_np(_POS_LEN, _D)[:_SEQ]


def _sc_body(table_hbm, idx_hbm, pos_hbm, out_hbm,
             idx0, idx1, g0, g1, g2, g3, posbuf,
             gs0, gs1, gs2, gs3, ss0, ss1, ss2, ss3, is0, is1):
    gbufs = (g0, g1, g2, g3)
    gsems = (gs0, gs1, gs2, gs3)
    ssems = (ss0, ss1, ss2, ss3)
    idxbufs = (idx0, idx1)
    isems = (is0, is1)

    wid = lax.axis_index("s") * _NC + lax.axis_index("c")
    ibase = wid * _IROWS_PER_W

    pltpu.sync_copy(pos_hbm, posbuf)

    def fire_idx(g, b):
        pltpu.async_copy(
            idx_hbm.at[pl.ds(ibase + 2 * g * _NG, 2 * _NG)],
            idxbufs[b], isems[b])

    def wait_idx(b):
        pltpu.make_async_copy(
            idx_hbm.at[pl.ds(ibase, 2 * _NG)], idxbufs[b], isems[b]
        ).wait()

    def do_fire(c_val, p, kpar, half):
        for j in range(_NG):
            pltpu.async_copy(
                table_hbm.at[idxbufs[kpar].at[half * _NG + j]],
                gbufs[p].at[pl.ds(j * _G, _G)], gsems[p])

    def wait_gather(p, kpar, half):
        for j in range(_NG):
            pltpu.make_async_copy(
                table_hbm.at[idxbufs[kpar].at[half * _NG + j]],
                gbufs[p].at[pl.ds(j * _G, _G)], gsems[p]).wait()

    def compute(p):
        gb = gbufs[p]

        def vec_body(t, carry):
            for h in range(2):
                pvs = [posbuf[h * _G + t, pl.ds(d * _LANES, _LANES)]
                       for d in range(_D // _LANES)]
                for q in range(2):
                    row = q * _SEQ + h * _G + t
                    for d in range(_D // _LANES):
                        col = d * _LANES
                        g = gb[row, pl.ds(col, _LANES)]
                        gb[row, pl.ds(col, _LANES)] = g * 8.0 + pvs[d]
            return carry

        lax.fori_loop(0, _G, vec_body, 0, unroll=2)

    def fire_scatter(c_val, p):
        o0 = wid * _N_CHUNKS + c_val
        pltpu.async_copy(
            gbufs[p], out_hbm.at[o0, :, pl.ds(0, _D)],
            ssems[p])

    def wait_scatter(p):
        pltpu.make_async_copy(
            gbufs[p], out_hbm.at[wid * _N_CHUNKS, :, pl.ds(0, _D)],
            ssems[p]).wait()

    def step(c_val, p, kpar, half, *, fire_next, wait_ssem,
             idx_wait_buf=None, idx_fire_group=None, idx_fire_buf=None):
        wait_gather(p, kpar, half)
        compute(p)
        fire_scatter(c_val, p)
        if fire_next:
            pn = (p + 2) % _NBUF
            kn = (kpar + 1) % 2
            if wait_ssem:
                wait_scatter(pn)
            if idx_wait_buf is not None:
                wait_idx(idx_wait_buf)
            do_fire(c_val + 2, pn, kn, half)
            if idx_fire_group is not None:
                fire_idx(idx_fire_group, idx_fire_buf)

    fire_idx(0, 0)
    fire_idx(1, 1)
    wait_idx(0)
    do_fire(0, 0, 0, 0)
    do_fire(1, 1, 0, 1)
    fire_idx(2, 0)
    wait_idx(1)

    step(0, 0, 0, 0, fire_next=True, wait_ssem=False)
    step(1, 1, 0, 1, fire_next=True, wait_ssem=False,
         idx_fire_group=3, idx_fire_buf=1)
    step(2, 2, 1, 0, fire_next=True, wait_ssem=True, idx_wait_buf=0)
    step(3, 3, 1, 1, fire_next=True, wait_ssem=True,
         idx_fire_group=4, idx_fire_buf=0)

    def super_body(i, carry):
        c0 = i * _NBUF
        g0_ = 2 * i
        step(c0 + 0, 0, 0, 0, fire_next=True, wait_ssem=True,
             idx_wait_buf=1)
        step(c0 + 1, 1, 0, 1, fire_next=True, wait_ssem=True,
             idx_fire_group=g0_ + 3, idx_fire_buf=1)
        step(c0 + 2, 2, 1, 0, fire_next=True, wait_ssem=True,
             idx_wait_buf=0)
        step(c0 + 3, 3, 1, 1, fire_next=True, wait_ssem=True,
             idx_fire_group=g0_ + 4, idx_fire_buf=0)
        return carry

    lax.fori_loop(1, _N_CHUNKS // _NBUF - 2, super_body, 0)

    step(56, 0, 0, 0, fire_next=True, wait_ssem=True, idx_wait_buf=1)
    step(57, 1, 0, 1, fire_next=True, wait_ssem=True,
         idx_fire_group=31, idx_fire_buf=1)
    step(58, 2, 1, 0, fire_next=True, wait_ssem=True, idx_wait_buf=0)
    step(59, 3, 1, 1, fire_next=True, wait_ssem=True)

    step(60, 0, 0, 0, fire_next=True, wait_ssem=True, idx_wait_buf=1)
    step(61, 1, 0, 1, fire_next=True, wait_ssem=True)
    step(62, 2, 1, 0, fire_next=False, wait_ssem=False)
    step(63, 3, 1, 1, fire_next=False, wait_ssem=False)

    for p in range(_NBUF):
        wait_scatter(p)


@jax.jit
def _sc_call(table, xidx, pos):
    mesh = plsc.VectorSubcoreMesh(
        core_axis_name="c", subcore_axis_name="s"
    )
    kfn = pl.kernel(
        _sc_body,
        out_type=jax.ShapeDtypeStruct((_N_ROWS // _CH, _CH, 2 * _D),
                                      jnp.float32),
        mesh=mesh,
        scratch_types=[
            pltpu.VMEM((2 * _NG, _G), jnp.int32),
            pltpu.VMEM((2 * _NG, _G), jnp.int32),
            pltpu.VMEM((_CH, _D), jnp.float32),
            pltpu.VMEM((_CH, _D), jnp.float32),
            pltpu.VMEM((_CH, _D), jnp.float32),
            pltpu.VMEM((_CH, _D), jnp.float32),
            pltpu.VMEM((_SEQ, _D), jnp.float32),
            pltpu.SemaphoreType.DMA,
            pltpu.SemaphoreType.DMA,
            pltpu.SemaphoreType.DMA,
            pltpu.SemaphoreType.DMA,
            pltpu.SemaphoreType.DMA,
            pltpu.SemaphoreType.DMA,
            pltpu.SemaphoreType.DMA,
            pltpu.SemaphoreType.DMA,
            pltpu.SemaphoreType.DMA,
            pltpu.SemaphoreType.DMA,
        ],
        compiler_params=pltpu.CompilerParams(use_tc_tiling_on_sc=False),
    )
    return kfn(table, xidx, pos)


def kernel(x, table):
    xidx = x.reshape((_N_ROWS // _G, _G))
    pos = jnp.asarray(_POS)
    out = _sc_call(table, xidx, pos)
    return out.reshape((_BATCH, _SEQ, 2 * _D))[:, :, :_D]


# speedup vs baseline: 1.8610x; 1.0004x over previous
import jax
import jax.numpy as jnp
import numpy as np
from jax import lax
from jax.experimental import pallas as pl
from jax.experimental.pallas import tpu as pltpu
from jax.experimental.pallas import tpu_sc as plsc

_VOCAB = 1000000
_D = 64
_BATCH = 4096
_SEQ = 200
_POS_LEN = 2048

_NC = 2
_NS = 16
_NW = _NC * _NS
_LANES = 16

_N_ROWS = _BATCH * _SEQ
_ROWS_PER_W = _N_ROWS // _NW
_CH = 400
_G = 100
_NG = _CH // _G
_N_CHUNKS = _ROWS_PER_W // _CH
_NBUF = 4
_IROWS_PER_W = _ROWS_PER_W // _G


def _pos_encoding_np(length, depth):
    half = depth / 2
    positions = np.arange(length)[:, np.newaxis]
    depths = np.arange(half)[np.newaxis, :] / half
    angle_rates = 1 / 10000 ** depths
    angle_rads = positions * angle_rates
    return np.concatenate(
        [np.sin(angle_rads), np.cos(angle_rads)], axis=-1
    ).astype(np.float32)


_POS = _pos_encoding_np(_POS_LEN, _D)[:_SEQ]


def _sc_body(table_hbm, idx_hbm, pos_hbm, out_hbm,
             idx0, idx1, g0, g1, g2, g3, posbuf,
             gs0, gs1, gs2, gs3, ss0, ss1, ss2, ss3, is0, is1):
    gbufs = (g0, g1, g2, g3)
    gsems = (gs0, gs1, gs2, gs3)
    ssems = (ss0, ss1, ss2, ss3)
    idxbufs = (idx0, idx1)
    isems = (is0, is1)

    wid = lax.axis_index("s") * _NC + lax.axis_index("c")
    ibase = wid * _IROWS_PER_W

    pltpu.sync_copy(pos_hbm, posbuf)

    def fire_idx(g, b):
        pltpu.async_copy(
            idx_hbm.at[pl.ds(ibase + 2 * g * _NG, 2 * _NG)],
            idxbufs[b], isems[b])

    def wait_idx(b):
        pltpu.make_async_copy(
            idx_hbm.at[pl.ds(ibase, 2 * _NG)], idxbufs[b], isems[b]
        ).wait()

    def do_fire(c_val, p, kpar, half):
        for j in range(_NG):
            pltpu.async_copy(
                table_hbm.at[idxbufs[kpar].at[half * _NG + j]],
                gbufs[p].at[j], gsems[p])

    def wait_gather(p, kpar, half):
        for j in range(_NG):
            pltpu.make_async_copy(
                table_hbm.at[idxbufs[kpar].at[half * _NG + j]],
                gbufs[p].at[j], gsems[p]).wait()

    def compute(p):
        gb = gbufs[p]

        def vec_body(t, carry):
            for h in range(2):
                pvs = [posbuf[h * _G + t, pl.ds(d * _LANES, _LANES)]
                       for d in range(_D // _LANES)]
                for q in range(2):
                    row = 2 * q + h
                    for d in range(_D // _LANES):
                        col = d * _LANES
                        g = gb[row, t, pl.ds(col, _LANES)]
                        gb[row, t, pl.ds(col, _LANES)] = g * 8.0 + pvs[d]
            return carry

        lax.fori_loop(0, _G, vec_body, 0, unroll=2)

    def fire_scatter(c_val, p):
        o0 = ibase + c_val * _NG
        pltpu.async_copy(
            gbufs[p], out_hbm.at[pl.ds(o0, _NG), :, pl.ds(0, _D)],
            ssems[p])

    def wait_scatter(p):
        pltpu.make_async_copy(
            gbufs[p], out_hbm.at[pl.ds(ibase, _NG), :, pl.ds(0, _D)],
            ssems[p]).wait()

    def step(c_val, p, kpar, half, *, fire_next, wait_ssem,
             idx_wait_buf=None, idx_fire_group=None, idx_fire_buf=None):
        wait_gather(p, kpar, half)
        compute(p)
        fire_scatter(c_val, p)
        if fire_next:
            pn = (p + 2) % _NBUF
            kn = (kpar + 1) % 2
            if wait_ssem:
                wait_scatter(pn)
            if idx_wait_buf is not None:
                wait_idx(idx_wait_buf)
            do_fire(c_val + 2, pn, kn, half)
            if idx_fire_group is not None:
                fire_idx(idx_fire_group, idx_fire_buf)

    fire_idx(0, 0)
    fire_idx(1, 1)
    wait_idx(0)
    do_fire(0, 0, 0, 0)
    do_fire(1, 1, 0, 1)
    wait_idx(1)

    step(0, 0, 0, 0, fire_next=True, wait_ssem=False)
    step(1, 1, 0, 1, fire_next=True, wait_ssem=False,
         idx_fire_group=2, idx_fire_buf=0)
    step(2, 2, 1, 0, fire_next=True, wait_ssem=True, idx_wait_buf=0)
    step(3, 3, 1, 1, fire_next=True, wait_ssem=True,
         idx_fire_group=3, idx_fire_buf=1)

    def super_body(i, carry):
        c0 = i * _NBUF
        g0_ = 2 * i
        step(c0 + 0, 0, 0, 0, fire_next=True, wait_ssem=True,
             idx_wait_buf=1)
        step(c0 + 1, 1, 0, 1, fire_next=True, wait_ssem=True,
             idx_fire_group=g0_ + 2, idx_fire_buf=0)
        step(c0 + 2, 2, 1, 0, fire_next=True, wait_ssem=True,
             idx_wait_buf=0)
        step(c0 + 3, 3, 1, 1, fire_next=True, wait_ssem=True,
             idx_fire_group=g0_ + 3, idx_fire_buf=1)
        return carry

    lax.fori_loop(1, _N_CHUNKS // _NBUF - 1, super_body, 0)

    step(60, 0, 0, 0, fire_next=True, wait_ssem=True, idx_wait_buf=1)
    step(61, 1, 0, 1, fire_next=True, wait_ssem=True)
    step(62, 2, 1, 0, fire_next=False, wait_ssem=False)
    step(63, 3, 1, 1, fire_next=False, wait_ssem=False)

    for p in range(_NBUF):
        wait_scatter(p)


@jax.jit
def _sc_call(table, xidx, pos):
    mesh = plsc.VectorSubcoreMesh(
        core_axis_name="c", subcore_axis_name="s"
    )
    kfn = pl.kernel(
        _sc_body,
        out_type=jax.ShapeDtypeStruct((_N_ROWS // _G, _G, 2 * _D),
                                      jnp.float32),
        mesh=mesh,
        scratch_types=[
            pltpu.VMEM((2 * _NG, _G), jnp.int32),
            pltpu.VMEM((2 * _NG, _G), jnp.int32),
            pltpu.VMEM((_NG, _G, _D), jnp.float32),
            pltpu.VMEM((_NG, _G, _D), jnp.float32),
            pltpu.VMEM((_NG, _G, _D), jnp.float32),
            pltpu.VMEM((_NG, _G, _D), jnp.float32),
            pltpu.VMEM((_SEQ, _D), jnp.float32),
            pltpu.SemaphoreType.DMA,
            pltpu.SemaphoreType.DMA,
            pltpu.SemaphoreType.DMA,
            pltpu.SemaphoreType.DMA,
            pltpu.SemaphoreType.DMA,
            pltpu.SemaphoreType.DMA,
            pltpu.SemaphoreType.DMA,
            pltpu.SemaphoreType.DMA,
            pltpu.SemaphoreType.DMA,
            pltpu.SemaphoreType.DMA,
        ],
        compiler_params=pltpu.CompilerParams(use_tc_tiling_on_sc=False),
    )
    return kfn(table, xidx, pos)


def kernel(x, table):
    xidx = x.reshape((_N_ROWS // _G, _G))
    pos = jnp.asarray(_POS)
    out = _sc_call(table, xidx, pos)
    return out.reshape((_BATCH, _SEQ, 2 * _D))[:, :, :_D]
